# initial kernel scaffold (unmeasured)
import jax
import jax.numpy as jnp
from jax import lax
from jax.experimental import pallas as pl
from jax.experimental.pallas import tpu as pltpu

N_DEV = 32


def kernel(x, w_mat):
    m, k_shard = x.shape
    _, n = w_mat.shape
    m_per = m // N_DEV
    nh = n // 2

    def body(x_ref, w_ref, out_ref, commR, commL,
             send_semR, recv_semR, send_semL, recv_semL,
             creditR, creditL):
        my = lax.axis_index("i")
        left = lax.rem(my + N_DEV - 1, N_DEV)
        right = lax.rem(my + 1, N_DEV)

        def partial(dest, lo, hi):
            xa = x_ref[pl.ds(dest * m_per, m_per), :]
            return jnp.dot(xa, w_ref[:, lo:hi],
                           preferred_element_type=jnp.float32)

        barrier = pltpu.get_barrier_semaphore()
        for nbr in (left, right):
            pl.semaphore_signal(barrier, inc=1, device_id=(nbr,),
                                device_id_type=pl.DeviceIdType.MESH)
        pl.semaphore_wait(barrier, 2)

        commR[0] = partial(left, 0, nh)
        commL[0] = partial(right, nh, n)

        for s in range(N_DEV - 1):
            snd, rcv = s % 2, (s + 1) % 2

            if s >= 2:
                pl.semaphore_wait(creditR, 1)
            rdmaR = pltpu.make_async_remote_copy(
                src_ref=commR.at[snd], dst_ref=commR.at[rcv],
                send_sem=send_semR.at[snd], recv_sem=recv_semR.at[rcv],
                device_id=(right,), device_id_type=pl.DeviceIdType.MESH)
            rdmaR.start()
            if s >= 2:
                pl.semaphore_wait(creditL, 1)
            rdmaL = pltpu.make_async_remote_copy(
                src_ref=commL.at[snd], dst_ref=commL.at[rcv],
                send_sem=send_semL.at[snd], recv_sem=recv_semL.at[rcv],
                device_id=(left,), device_id_type=pl.DeviceIdType.MESH)
            rdmaL.start()

            pR = partial(lax.rem(my + 2 * N_DEV - 2 - s, N_DEV), 0, nh)
            pL = partial(lax.rem(my + 2 + s, N_DEV), nh, n)

            rdmaR.wait_send()
            rdmaL.wait_send()
            if 1 <= s <= N_DEV - 3:
                pl.semaphore_signal(creditR, inc=1, device_id=(left,),
                                    device_id_type=pl.DeviceIdType.MESH)
                pl.semaphore_signal(creditL, inc=1, device_id=(right,),
                                    device_id_type=pl.DeviceIdType.MESH)

            rdmaR.wait_recv()
            commR[rcv] = commR[rcv] + pR
            rdmaL.wait_recv()
            commL[rcv] = commL[rcv] + pL

        fin = (N_DEV - 1) % 2
        yR = commR[fin]
        yL = commL[fin]
        out_ref[:, 0:nh] = yR * jax.nn.sigmoid(yR)
        out_ref[:, nh:n] = yL * jax.nn.sigmoid(yL)

    return pl.pallas_call(
        body,
        out_shape=jax.ShapeDtypeStruct((m_per, n), jnp.float32),
        in_specs=[pl.BlockSpec(memory_space=pltpu.VMEM),
                  pl.BlockSpec(memory_space=pltpu.VMEM)],
        out_specs=pl.BlockSpec(memory_space=pltpu.VMEM),
        scratch_shapes=[
            pltpu.VMEM((2, m_per, nh), jnp.float32),
            pltpu.VMEM((2, m_per, nh), jnp.float32),
            pltpu.SemaphoreType.DMA((2,)),
            pltpu.SemaphoreType.DMA((2,)),
            pltpu.SemaphoreType.DMA((2,)),
            pltpu.SemaphoreType.DMA((2,)),
            pltpu.SemaphoreType.REGULAR,
            pltpu.SemaphoreType.REGULAR,
        ],
        compiler_params=pltpu.CompilerParams(collective_id=0),
    )(x, w_mat)


# baseline (device time: 1490795 ns/iter reference)
import jax
import jax.numpy as jnp
from jax import lax
from jax.experimental import pallas as pl
from jax.experimental.pallas import tpu as pltpu

N_DEV = 32


def kernel(x, w_mat):
    m, k_shard = x.shape
    _, n = w_mat.shape
    m_per = m // N_DEV
    nh = n // 2

    def body(x_ref, w_ref, out_ref, commR, commL,
             send_semR, recv_semR, send_semL, recv_semL,
             creditR, creditL):
        my = lax.axis_index("i")
        left = lax.rem(my + N_DEV - 1, N_DEV)
        right = lax.rem(my + 1, N_DEV)

        def partial(dest, lo, hi):
            xa = x_ref[pl.ds(dest * m_per, m_per), :]
            return jnp.dot(xa, w_ref[:, lo:hi],
                           preferred_element_type=jnp.float32)

        barrier = pltpu.get_barrier_semaphore()
        for nbr in (left, right):
            pl.semaphore_signal(barrier, inc=1, device_id=(nbr,),
                                device_id_type=pl.DeviceIdType.MESH)
        pl.semaphore_wait(barrier, 2)

        commR[0] = partial(left, 0, nh)
        commL[0] = partial(right, nh, n)

        for s in range(N_DEV - 1):
            snd, rcv = s % 2, (s + 1) % 2

            if s >= 1:
                pl.semaphore_wait(creditR, 1)
            rdmaR = pltpu.make_async_remote_copy(
                src_ref=commR.at[snd], dst_ref=commR.at[rcv],
                send_sem=send_semR.at[snd], recv_sem=recv_semR.at[rcv],
                device_id=(right,), device_id_type=pl.DeviceIdType.MESH)
            rdmaR.start()
            if s >= 1:
                pl.semaphore_wait(creditL, 1)
            rdmaL = pltpu.make_async_remote_copy(
                src_ref=commL.at[snd], dst_ref=commL.at[rcv],
                send_sem=send_semL.at[snd], recv_sem=recv_semL.at[rcv],
                device_id=(left,), device_id_type=pl.DeviceIdType.MESH)
            rdmaL.start()

            pR = partial(lax.rem(my + 2 * N_DEV - 2 - s, N_DEV), 0, nh)
            pL = partial(lax.rem(my + 2 + s, N_DEV), nh, n)

            rdmaR.wait_send()
            rdmaL.wait_send()
            if s <= N_DEV - 3:
                pl.semaphore_signal(creditR, inc=1, device_id=(left,),
                                    device_id_type=pl.DeviceIdType.MESH)
                pl.semaphore_signal(creditL, inc=1, device_id=(right,),
                                    device_id_type=pl.DeviceIdType.MESH)

            rdmaR.wait_recv()
            commR[rcv] = commR[rcv] + pR
            rdmaL.wait_recv()
            commL[rcv] = commL[rcv] + pL

        fin = (N_DEV - 1) % 2
        yR = commR[fin]
        yL = commL[fin]
        out_ref[:, 0:nh] = yR * jax.nn.sigmoid(yR)
        out_ref[:, nh:n] = yL * jax.nn.sigmoid(yL)

    return pl.pallas_call(
        body,
        out_shape=jax.ShapeDtypeStruct((m_per, n), jnp.float32),
        in_specs=[pl.BlockSpec(memory_space=pltpu.VMEM),
                  pl.BlockSpec(memory_space=pltpu.VMEM)],
        out_specs=pl.BlockSpec(memory_space=pltpu.VMEM),
        scratch_shapes=[
            pltpu.VMEM((2, m_per, nh), jnp.float32),
            pltpu.VMEM((2, m_per, nh), jnp.float32),
            pltpu.SemaphoreType.DMA((2,)),
            pltpu.SemaphoreType.DMA((2,)),
            pltpu.SemaphoreType.DMA((2,)),
            pltpu.SemaphoreType.DMA((2,)),
            pltpu.SemaphoreType.REGULAR,
            pltpu.SemaphoreType.REGULAR,
        ],
        compiler_params=pltpu.CompilerParams(collective_id=0),
    )(x, w_mat)


# device time: 776057 ns/iter; 1.9210x vs baseline; 1.9210x over previous
import jax
import jax.numpy as jnp
from jax import lax
from jax.experimental import pallas as pl
from jax.experimental.pallas import tpu as pltpu

N_DEV = 32

PERM = [0, 3, 4, 7, 15, 12, 11, 8, 16, 19, 20, 23, 31, 28, 27, 24,
        25, 26, 29, 30, 22, 21, 18, 17, 9, 10, 13, 14, 6, 5, 2, 1]


def kernel(x, w_mat):
    m, k_shard = x.shape
    _, n = w_mat.shape
    m_per = m // N_DEV
    nh = n // 2

    def body(perm_ref, inv_ref, x_ref, w_ref, out_ref, commR, commL,
             send_semR, recv_semR, send_semL, recv_semL,
             creditR, creditL):
        my = lax.axis_index("i")

        r = inv_ref[0, my]

        def perm_at(pos):
            return perm_ref[0, lax.rem(pos, N_DEV)]

        left = perm_at(r + N_DEV - 1)
        right = perm_at(r + 1)

        def partial(dest, lo, hi):
            xa = x_ref[pl.ds(dest * m_per, m_per), :]
            return jnp.dot(xa, w_ref[:, lo:hi],
                           preferred_element_type=jnp.float32)

        barrier = pltpu.get_barrier_semaphore()
        for nbr in (left, right):
            pl.semaphore_signal(barrier, inc=1, device_id=(nbr,),
                                device_id_type=pl.DeviceIdType.MESH)
        pl.semaphore_wait(barrier, 2)

        commR[0] = partial(left, 0, nh)
        commL[0] = partial(right, nh, n)

        for s in range(N_DEV - 1):
            snd, rcv = s % 2, (s + 1) % 2

            if s >= 1:
                pl.semaphore_wait(creditR, 1)
            rdmaR = pltpu.make_async_remote_copy(
                src_ref=commR.at[snd], dst_ref=commR.at[rcv],
                send_sem=send_semR.at[snd], recv_sem=recv_semR.at[rcv],
                device_id=(right,), device_id_type=pl.DeviceIdType.MESH)
            rdmaR.start()
            if s >= 1:
                pl.semaphore_wait(creditL, 1)
            rdmaL = pltpu.make_async_remote_copy(
                src_ref=commL.at[snd], dst_ref=commL.at[rcv],
                send_sem=send_semL.at[snd], recv_sem=recv_semL.at[rcv],
                device_id=(left,), device_id_type=pl.DeviceIdType.MESH)
            rdmaL.start()

            pR = partial(perm_at(r + 2 * N_DEV - 2 - s), 0, nh)
            pL = partial(perm_at(r + 2 + s), nh, n)

            rdmaR.wait_send()
            rdmaL.wait_send()
            if s <= N_DEV - 3:
                pl.semaphore_signal(creditR, inc=1, device_id=(left,),
                                    device_id_type=pl.DeviceIdType.MESH)
                pl.semaphore_signal(creditL, inc=1, device_id=(right,),
                                    device_id_type=pl.DeviceIdType.MESH)

            rdmaR.wait_recv()
            commR[rcv] = commR[rcv] + pR
            rdmaL.wait_recv()
            commL[rcv] = commL[rcv] + pL

        fin = (N_DEV - 1) % 2
        yR = commR[fin]
        yL = commL[fin]
        out_ref[:, 0:nh] = yR * jax.nn.sigmoid(yR)
        out_ref[:, nh:n] = yL * jax.nn.sigmoid(yL)

    perm = jnp.array(PERM, dtype=jnp.int32).reshape(1, N_DEV)
    inv = jnp.zeros(N_DEV, dtype=jnp.int32).at[jnp.array(PERM)].set(
        jnp.arange(N_DEV, dtype=jnp.int32)).reshape(1, N_DEV)

    return pl.pallas_call(
        body,
        out_shape=jax.ShapeDtypeStruct((m_per, n), jnp.float32),
        in_specs=[pl.BlockSpec(memory_space=pltpu.SMEM),
                  pl.BlockSpec(memory_space=pltpu.SMEM),
                  pl.BlockSpec(memory_space=pltpu.VMEM),
                  pl.BlockSpec(memory_space=pltpu.VMEM)],
        out_specs=pl.BlockSpec(memory_space=pltpu.VMEM),
        scratch_shapes=[
            pltpu.VMEM((2, m_per, nh), jnp.float32),
            pltpu.VMEM((2, m_per, nh), jnp.float32),
            pltpu.SemaphoreType.DMA((2,)),
            pltpu.SemaphoreType.DMA((2,)),
            pltpu.SemaphoreType.DMA((2,)),
            pltpu.SemaphoreType.DMA((2,)),
            pltpu.SemaphoreType.REGULAR,
            pltpu.SemaphoreType.REGULAR,
        ],
        compiler_params=pltpu.CompilerParams(collective_id=0),
    )(perm, inv, x, w_mat)
